# Initial kernel scaffold; baseline (speedup 1.0000x reference)
import jax
import jax.numpy as jnp
from jax.experimental import pallas as pl


def kernel(global_context, piece_context, flags, consequence, square_emb, kind_emb, promo_emb, meta_emb, null_piece, fW1, fb1, fW2, fb2, cW1, cb1, cW2, cb2, ln_g, ln_b, oW1, ob1, oW2, ob2, moving_idx, target_idx, from_sq, to_sq, move_kind, promo_idx, meta_idx):
    raise NotImplementedError("write your pallas kernel here")


# speedup vs baseline: 1.7474x; 1.7474x over previous
import functools

import jax
import jax.numpy as jnp
from jax import lax
from jax.experimental import pallas as pl
from jax.experimental.pallas import tpu as pltpu
from jax.experimental.pallas import tpu_sc as plsc

N = 16384
D = 192
H = 384
P = 64
N_SQ = 65
N_KIND = 49
N_PROMO = 5
N_META = 129
T_ROWS = (P + 1) + N_SQ + N_KIND + N_PROMO + N_META

NC = 2
NS = 16
NW = NC * NS
RPW = N // NW
C = 64
NCHUNK = RPW // C
NVEC = D // 16


def _gelu(x):
    return 0.5 * x * (1.0 + lax.erf(x * 0.7071067811865476))


def _sc_gather_sum(table, idx):
    mesh = plsc.VectorSubcoreMesh(core_axis_name="c", subcore_axis_name="s")

    @functools.partial(
        pl.kernel,
        mesh=mesh,
        out_type=jax.ShapeDtypeStruct((N, D), jnp.float32),
        scratch_types=[
            pltpu.VMEM((7, RPW), jnp.int32),
            pltpu.VMEM((C, D), jnp.float32),
            pltpu.VMEM((C, D), jnp.float32),
            pltpu.SemaphoreType.DMA,
        ],
        compiler_params=pltpu.CompilerParams(use_tc_tiling_on_sc=False),
    )
    def k(table_hbm, idx_hbm, out_hbm, idx_v, acc_v, buf_v, sem):
        wid = lax.axis_index("s") * NC + lax.axis_index("c")
        base = wid * RPW
        pltpu.sync_copy(idx_hbm.at[wid], idx_v)

        def chunk(ci, _):
            row0 = base + ci * C
            col0 = ci * C
            pltpu.async_copy(
                table_hbm.at[idx_v.at[0, pl.ds(col0, C)]], acc_v, sem).wait()
            for t in range(1, 7):
                pltpu.async_copy(
                    table_hbm.at[idx_v.at[t, pl.ds(col0, C)]], buf_v,
                    sem).wait()

                def add_row(i, _):
                    for kk in range(NVEC):
                        sl = pl.ds(kk * 16, 16)
                        acc_v[i, sl] = acc_v[i, sl] + buf_v[i, sl]
                    return 0

                lax.fori_loop(0, C, add_row, 0, unroll=2)
            pltpu.sync_copy(acc_v, out_hbm.at[pl.ds(row0, C)])
            return 0

        lax.fori_loop(0, NCHUNK, chunk, 0)

    return k(table, idx)


def _tc_body(tok_ref, flags_ref, cons_ref, gctx_ref, fW1_ref, fb1_ref,
             fW2_ref, fb2_ref, cW1_ref, cb1_ref, cW2_ref, cb2_ref,
             ln_g_ref, ln_b_ref, oW1_ref, ob1_ref, oW2_ref, ob2_ref,
             out_ref):
    f32 = jnp.float32
    tok = tok_ref[...] + gctx_ref[...]
    h1 = _gelu(jnp.dot(flags_ref[...], fW1_ref[...],
                       preferred_element_type=f32) + fb1_ref[...])
    tok = tok + jnp.dot(h1, fW2_ref[...], preferred_element_type=f32) + fb2_ref[...]
    h2 = _gelu(jnp.dot(cons_ref[...], cW1_ref[...],
                       preferred_element_type=f32) + cb1_ref[...])
    tok = tok + jnp.dot(h2, cW2_ref[...], preferred_element_type=f32) + cb2_ref[...]
    mu = jnp.mean(tok, axis=-1, keepdims=True)
    cen = tok - mu
    var = jnp.mean(cen * cen, axis=-1, keepdims=True)
    h = cen * jax.lax.rsqrt(var + 1e-5) * ln_g_ref[...] + ln_b_ref[...]
    h3 = _gelu(jnp.dot(h, oW1_ref[...], preferred_element_type=f32) + ob1_ref[...])
    out_ref[...] = (jnp.dot(h3, oW2_ref[...], preferred_element_type=f32)
                    + ob2_ref[...])


def _tc_encode(tok, flags, cons, gctx, fW1, fb1, fW2, fb2, cW1, cb1, cW2,
               cb2, ln_g, ln_b, oW1, ob1, oW2, ob2, block_n=2048):
    grid = (N // block_n,)

    def rows(bn):
        return pl.BlockSpec((bn, None), lambda i: (i, 0))

    def full(shape):
        return pl.BlockSpec(shape, lambda i: tuple(0 for _ in shape))

    rows_spec = pl.BlockSpec((block_n, D), lambda i: (i, 0))
    in_specs = [
        rows_spec,
        pl.BlockSpec((block_n, 7), lambda i: (i, 0)),
        pl.BlockSpec((block_n, 12), lambda i: (i, 0)),
        full((1, D)),
        full((7, H)), full((1, H)), full((H, D)), full((1, D)),
        full((12, H)), full((1, H)), full((H, D)), full((1, D)),
        full((1, D)), full((1, D)),
        full((D, H)), full((1, H)), full((H, D)), full((1, D)),
    ]
    return pl.pallas_call(
        _tc_body,
        grid=grid,
        in_specs=in_specs,
        out_specs=rows_spec,
        out_shape=jax.ShapeDtypeStruct((N, D), jnp.float32),
    )(tok, flags, cons, gctx, fW1, fb1, fW2, fb2, cW1, cb1, cW2, cb2,
      ln_g, ln_b, oW1, ob1, oW2, ob2)


def kernel(global_context, piece_context, flags, consequence, square_emb,
           kind_emb, promo_emb, meta_emb, null_piece, fW1, fb1, fW2, fb2,
           cW1, cb1, cW2, cb2, ln_g, ln_b, oW1, ob1, oW2, ob2, moving_idx,
           target_idx, from_sq, to_sq, move_kind, promo_idx, meta_idx):
    piece_full = jnp.concatenate([piece_context, null_piece[None, :]], axis=0)
    table = jnp.concatenate(
        [piece_full, square_emb, kind_emb, promo_emb, meta_emb], axis=0)
    o_sq = P + 1
    o_kind = o_sq + N_SQ
    o_promo = o_kind + N_KIND
    o_meta = o_promo + N_PROMO
    i32 = jnp.int32
    idx = jnp.stack([
        moving_idx.astype(i32),
        target_idx.astype(i32),
        from_sq.astype(i32) + o_sq,
        to_sq.astype(i32) + o_sq,
        move_kind.astype(i32) + o_kind,
        promo_idx.astype(i32) + o_promo,
        meta_idx.astype(i32) + o_meta,
    ])
    idx = idx.reshape(7, NW, RPW).transpose(1, 0, 2)

    tok = _sc_gather_sum(table, idx)

    r1 = lambda v: v[None, :]
    return _tc_encode(tok, flags, consequence, r1(global_context),
                      fW1, r1(fb1), fW2, r1(fb2),
                      cW1, r1(cb1), cW2, r1(cb2),
                      r1(ln_g), r1(ln_b),
                      oW1, r1(ob1), oW2, r1(ob2))


# speedup vs baseline: 2.2849x; 1.3076x over previous
import functools

import jax
import jax.numpy as jnp
from jax import lax
from jax.experimental import pallas as pl
from jax.experimental.pallas import tpu as pltpu
from jax.experimental.pallas import tpu_sc as plsc

N = 16384
D = 192
H = 384
P = 64
N_SQ = 65
N_KIND = 49
N_PROMO = 5
N_META = 129
T_ROWS = (P + 1) + N_SQ + N_KIND + N_PROMO + N_META

NC = 2
NS = 16
NW = NC * NS
RPW = N // NW
C = 128
NCHUNK = RPW // C
NVEC = D // 16


def _gelu(x):
    return 0.5 * x * (1.0 + lax.erf(x * 0.7071067811865476))


def _sc_gather_sum(table, idx):
    mesh = plsc.VectorSubcoreMesh(core_axis_name="c", subcore_axis_name="s")

    @functools.partial(
        pl.kernel,
        mesh=mesh,
        out_type=jax.ShapeDtypeStruct((N, D), jnp.float32),
        scratch_types=[
            pltpu.VMEM((7, RPW), jnp.int32),
            pltpu.VMEM((C, D), jnp.float32),
            pltpu.SemaphoreType.DMA,
        ],
        compiler_params=pltpu.CompilerParams(use_tc_tiling_on_sc=False),
    )
    def k(table_hbm, idx_hbm, out_hbm, idx_v, acc_v, sem):
        wid = lax.axis_index("s") * NC + lax.axis_index("c")
        base = wid * RPW
        pltpu.sync_copy(idx_hbm.at[wid], idx_v)

        def chunk(ci, _):
            row0 = base + ci * C
            col0 = ci * C
            pltpu.async_copy(
                table_hbm.at[idx_v.at[0, pl.ds(col0, C)]], acc_v, sem).wait()
            cps = [
                pltpu.async_copy(
                    table_hbm.at[idx_v.at[t, pl.ds(col0, C)]], acc_v, sem,
                    add=True)
                for t in range(1, 7)
            ]
            for cp in cps:
                cp.wait()
            pltpu.sync_copy(acc_v, out_hbm.at[pl.ds(row0, C)])
            return 0

        lax.fori_loop(0, NCHUNK, chunk, 0)

    return k(table, idx)


def _tc_body(tok_ref, flags_ref, cons_ref, gctx_ref, fW1_ref, fb1_ref,
             fW2_ref, fb2_ref, cW1_ref, cb1_ref, cW2_ref, cb2_ref,
             ln_g_ref, ln_b_ref, oW1_ref, ob1_ref, oW2_ref, ob2_ref,
             out_ref):
    f32 = jnp.float32
    tok = tok_ref[...] + gctx_ref[...]
    h1 = _gelu(jnp.dot(flags_ref[...], fW1_ref[...],
                       preferred_element_type=f32) + fb1_ref[...])
    tok = tok + jnp.dot(h1, fW2_ref[...], preferred_element_type=f32) + fb2_ref[...]
    h2 = _gelu(jnp.dot(cons_ref[...], cW1_ref[...],
                       preferred_element_type=f32) + cb1_ref[...])
    tok = tok + jnp.dot(h2, cW2_ref[...], preferred_element_type=f32) + cb2_ref[...]
    mu = jnp.mean(tok, axis=-1, keepdims=True)
    cen = tok - mu
    var = jnp.mean(cen * cen, axis=-1, keepdims=True)
    h = cen * jax.lax.rsqrt(var + 1e-5) * ln_g_ref[...] + ln_b_ref[...]
    h3 = _gelu(jnp.dot(h, oW1_ref[...], preferred_element_type=f32) + ob1_ref[...])
    out_ref[...] = (jnp.dot(h3, oW2_ref[...], preferred_element_type=f32)
                    + ob2_ref[...])


def _tc_encode(tok, flags, cons, gctx, fW1, fb1, fW2, fb2, cW1, cb1, cW2,
               cb2, ln_g, ln_b, oW1, ob1, oW2, ob2, block_n=2048):
    grid = (N // block_n,)

    def rows(bn):
        return pl.BlockSpec((bn, None), lambda i: (i, 0))

    def full(shape):
        return pl.BlockSpec(shape, lambda i: tuple(0 for _ in shape))

    rows_spec = pl.BlockSpec((block_n, D), lambda i: (i, 0))
    in_specs = [
        rows_spec,
        pl.BlockSpec((block_n, 7), lambda i: (i, 0)),
        pl.BlockSpec((block_n, 12), lambda i: (i, 0)),
        full((1, D)),
        full((7, H)), full((1, H)), full((H, D)), full((1, D)),
        full((12, H)), full((1, H)), full((H, D)), full((1, D)),
        full((1, D)), full((1, D)),
        full((D, H)), full((1, H)), full((H, D)), full((1, D)),
    ]
    return pl.pallas_call(
        _tc_body,
        grid=grid,
        in_specs=in_specs,
        out_specs=rows_spec,
        out_shape=jax.ShapeDtypeStruct((N, D), jnp.float32),
    )(tok, flags, cons, gctx, fW1, fb1, fW2, fb2, cW1, cb1, cW2, cb2,
      ln_g, ln_b, oW1, ob1, oW2, ob2)


def kernel(global_context, piece_context, flags, consequence, square_emb,
           kind_emb, promo_emb, meta_emb, null_piece, fW1, fb1, fW2, fb2,
           cW1, cb1, cW2, cb2, ln_g, ln_b, oW1, ob1, oW2, ob2, moving_idx,
           target_idx, from_sq, to_sq, move_kind, promo_idx, meta_idx):
    piece_full = jnp.concatenate([piece_context, null_piece[None, :]], axis=0)
    table = jnp.concatenate(
        [piece_full, square_emb, kind_emb, promo_emb, meta_emb], axis=0)
    o_sq = P + 1
    o_kind = o_sq + N_SQ
    o_promo = o_kind + N_KIND
    o_meta = o_promo + N_PROMO
    i32 = jnp.int32
    idx = jnp.stack([
        moving_idx.astype(i32),
        target_idx.astype(i32),
        from_sq.astype(i32) + o_sq,
        to_sq.astype(i32) + o_sq,
        move_kind.astype(i32) + o_kind,
        promo_idx.astype(i32) + o_promo,
        meta_idx.astype(i32) + o_meta,
    ])
    idx = idx.reshape(7, NW, RPW).transpose(1, 0, 2)

    tok = _sc_gather_sum(table, idx)

    r1 = lambda v: v[None, :]
    return _tc_encode(tok, flags, consequence, r1(global_context),
                      fW1, r1(fb1), fW2, r1(fb2),
                      cW1, r1(cb1), cW2, r1(cb2),
                      r1(ln_g), r1(ln_b),
                      oW1, r1(ob1), oW2, r1(ob2))
